# baseline (speedup 1.0000x reference)
import jax
import jax.numpy as jnp
from jax.experimental import pallas as pl

Q = 2048
K = 16384
D = 256
TOPK = 256

TI = 256
TJ = 2048
NI = Q // TI
NJ = K // TJ


def _score_kernel(ref_ref, src_ref, s_ref, r_ref, c_ref):
    i = pl.program_id(0)
    j = pl.program_id(1)
    x = ref_ref[...]
    y = src_ref[...]
    xn = x / (jnp.sqrt(jnp.sum(x * x, axis=1, keepdims=True)) + 1e-12)
    yn = y / (jnp.sqrt(jnp.sum(y * y, axis=1, keepdims=True)) + 1e-12)
    x2 = jnp.sum(xn * xn, axis=1, keepdims=True)
    y2 = jnp.sum(yn * yn, axis=1, keepdims=True)
    xy = jnp.dot(xn, yn.T, preferred_element_type=jnp.float32)
    d = jnp.maximum(x2 - 2.0 * xy + y2.T, 0.0)
    s = jnp.exp(-d)
    s_ref[...] = s

    @pl.when(jnp.logical_and(i == 0, j == 0))
    def _():
        r_ref[...] = jnp.zeros_like(r_ref)
        c_ref[...] = jnp.zeros_like(c_ref)

    r_ref[pl.ds(i * TI, TI), :] += jnp.sum(s, axis=1, keepdims=True)
    c_ref[:, pl.ds(j * TJ, TJ)] += jnp.sum(s, axis=0, keepdims=True)


def _dualnorm_kernel(s_ref, r_ref, c_ref, b_ref):
    s = s_ref[...]
    b_ref[...] = (s / r_ref[...]) * (s / c_ref[...])


def kernel(ref_feats, src_feats):
    s, r, c = pl.pallas_call(
        _score_kernel,
        grid=(NI, NJ),
        in_specs=[
            pl.BlockSpec((TI, D), lambda i, j: (i, 0)),
            pl.BlockSpec((TJ, D), lambda i, j: (j, 0)),
        ],
        out_specs=[
            pl.BlockSpec((TI, TJ), lambda i, j: (i, j)),
            pl.BlockSpec((Q, 1), lambda i, j: (0, 0)),
            pl.BlockSpec((1, K), lambda i, j: (0, 0)),
        ],
        out_shape=[
            jax.ShapeDtypeStruct((Q, K), jnp.float32),
            jax.ShapeDtypeStruct((Q, 1), jnp.float32),
            jax.ShapeDtypeStruct((1, K), jnp.float32),
        ],
    )(ref_feats, src_feats)

    b = pl.pallas_call(
        _dualnorm_kernel,
        grid=(NI, NJ),
        in_specs=[
            pl.BlockSpec((TI, TJ), lambda i, j: (i, j)),
            pl.BlockSpec((TI, 1), lambda i, j: (i, 0)),
            pl.BlockSpec((1, TJ), lambda i, j: (0, j)),
        ],
        out_specs=pl.BlockSpec((TI, TJ), lambda i, j: (i, j)),
        out_shape=jax.ShapeDtypeStruct((Q, K), jnp.float32),
    )(s, r, c)

    corr_scores, corr_indices = jax.lax.top_k(b.reshape(-1), TOPK)
    ref_corr_indices = corr_indices // K
    src_corr_indices = corr_indices % K
    return corr_scores, ref_corr_indices, src_corr_indices
